# Initial kernel scaffold; baseline (speedup 1.0000x reference)
import jax
import jax.numpy as jnp
from jax.experimental import pallas as pl


def kernel(route_logits, scale_logits, uncertainty_logits, image_size):
    raise NotImplementedError("write your pallas kernel here")


# speedup vs baseline: 2.3681x; 2.3681x over previous
import jax
import jax.numpy as jnp
from jax import lax
from jax.experimental import pallas as pl
from jax.experimental.pallas import tpu as pltpu

B, H, W = 4, 256, 256
N = H * W
C = 64
L = 1024
K = 1000
STRIDE = 4.0
MIN_SIZE = 32.0
MAX_SIZE = 512.0


def _lane_iota():
    return lax.broadcasted_iota(jnp.int32, (1, 1, L), 2)


def _cmp_exchange(v, ix, d, dir_desc):
    j = _lane_iota()
    first = (j & d) == 0
    keep_max = dir_desc == first
    vu = jnp.roll(v, -d, axis=-1)
    vd = jnp.roll(v, d, axis=-1)
    iu = jnp.roll(ix, -d, axis=-1)
    idn = jnp.roll(ix, d, axis=-1)
    pv = jnp.where(first, vu, vd)
    pi = jnp.where(first, iu, idn)
    g = (v > pv) | ((v == pv) & (ix < pi))
    take_self = g == keep_max
    return jnp.where(take_self, v, pv), jnp.where(take_self, ix, pi)


def _sort_rows(v, ix, row_desc):
    j = _lane_iota()
    for p in range(1, 11):
        blk_desc = ((j >> p) & 1) == 0
        dir_desc = blk_desc == row_desc
        for s in range(p - 1, -1, -1):
            v, ix = _cmp_exchange(v, ix, 1 << s, dir_desc)
    return v, ix


def _bitonic_merge(v, ix, row_desc):
    for s in range(9, -1, -1):
        v, ix = _cmp_exchange(v, ix, 1 << s, row_desc)
    return v, ix


def _row_desc_mask(m):
    r = lax.broadcasted_iota(jnp.int32, (1, m, 1), 1)
    return r < (m // 2) if m > 1 else r < 1


def _topk_kernel(route_ref, unc_ref, scores_ref, idx_ref):
    x = route_ref[...]
    u = unc_ref[...]
    s = jax.nn.sigmoid(x)
    score = s * s * (1.0 - 0.35 * jax.nn.sigmoid(u))
    fill_row = jnp.full((B, 1, W), -1.0, jnp.float32)
    up = jnp.concatenate([score[:, 1:, :], fill_row], axis=1)
    dn = jnp.concatenate([fill_row, score[:, :-1, :]], axis=1)
    vert = jnp.maximum(score, jnp.maximum(up, dn))
    fill_col = jnp.full((B, H, 1), -1.0, jnp.float32)
    lf = jnp.concatenate([vert[:, :, 1:], fill_col], axis=2)
    rt = jnp.concatenate([fill_col, vert[:, :, :-1]], axis=2)
    pooled = jnp.maximum(vert, jnp.maximum(lf, rt))
    filt = jnp.where(score == pooled, score, -1.0)

    v = filt.reshape(B, C, L)
    ci = lax.broadcasted_iota(jnp.int32, (B, C, L), 1)
    ji = lax.broadcasted_iota(jnp.int32, (B, C, L), 2)
    ix = ci * L + ji

    v, ix = _sort_rows(v, ix, _row_desc_mask(C))
    m = C
    while m > 1:
        half = m // 2
        av, bv = v[:, :half, :], v[:, half:, :]
        ai, bi = ix[:, :half, :], ix[:, half:, :]
        g = (av > bv) | ((av == bv) & (ai < bi))
        v = jnp.where(g, av, bv)
        ix = jnp.where(g, ai, bi)
        v, ix = _bitonic_merge(v, ix, _row_desc_mask(half))
        m = half

    vf = v.reshape(B, L)
    scores_ref[...] = jnp.where(vf > 0.0, vf, 0.0)
    idx_ref[...] = ix.reshape(B, L)


def _run_topk(route, unc):
    return pl.pallas_call(
        _topk_kernel,
        out_shape=[
            jax.ShapeDtypeStruct((B, L), jnp.float32),
            jax.ShapeDtypeStruct((B, L), jnp.int32),
        ],
    )(route, unc)


def kernel(route_logits, scale_logits, uncertainty_logits, image_size):
    route = route_logits.reshape(B, H, W)
    unc = uncertainty_logits.reshape(B, H, W)
    scores, idx = _run_topk(route, unc)

    image_h = image_size[0].astype(jnp.float32)
    image_w = image_size[1].astype(jnp.float32)
    ys = idx // W
    xs = idx % W
    cx = (xs.astype(jnp.float32) + 0.5) * STRIDE
    cy = (ys.astype(jnp.float32) + 0.5) * STRIDE
    scale_flat = scale_logits.reshape(B, -1)
    unc_flat = uncertainty_logits.reshape(B, -1)
    scale_g = jnp.take_along_axis(scale_flat, idx, axis=1)
    unc_g = jax.nn.sigmoid(jnp.take_along_axis(unc_flat, idx, axis=1))
    side = MIN_SIZE + jax.nn.sigmoid(scale_g) * (MAX_SIZE - MIN_SIZE)
    side = side * (1.0 + 0.25 * unc_g)
    half = side / 2.0
    x1 = jnp.clip(cx - half, 0.0, image_w - 1.0)
    y1 = jnp.clip(cy - half, 0.0, image_h - 1.0)
    x2 = jnp.clip(cx + half, 1.0, image_w)
    y2 = jnp.clip(cy + half, 1.0, image_h)
    bidx = jnp.broadcast_to(jnp.arange(B, dtype=jnp.float32)[:, None], (B, L))
    rois = jnp.stack([bidx, x1, y1, x2, y2], axis=2)
    vm = (scores > 0.0).astype(jnp.float32)
    rois = rois * vm[:, :, None]
    rois = rois[:, :K, :].reshape(B * K, 5)
    out_scores = scores[:, :K].reshape(B * K)
    return rois, out_scores
